# Initial kernel scaffold; baseline (speedup 1.0000x reference)
import jax
import jax.numpy as jnp
from jax.experimental import pallas as pl


def kernel(x_sparse, x_dense, tables, W_wide, b_wide, W1, b1, W2, b2, W3, b3, mix):
    raise NotImplementedError("write your pallas kernel here")


# speedup vs baseline: 2.0063x; 2.0063x over previous
import functools

import jax
import jax.numpy as jnp
from jax import lax
from jax.experimental import pallas as pl
from jax.experimental.pallas import tpu as pltpu
from jax.experimental.pallas import tpu_sc as plsc

B = 16384
F_SPARSE = 26
EMB = 16
VOCAB = 100000
DENSE = 13
SE_DIM = F_SPARSE * EMB

NC = 2
NS = 16
NW = NC * NS
N_ROWS = B * F_SPARSE
PER_W = N_ROWS // NW
CHUNK = 128
N_CHUNKS = PER_W // CHUNK


def _sc_gather_body(idx_hbm, tab_hbm, out_hbm, idx_v, rows_v, gsem):
    wid = lax.axis_index("s") * NC + lax.axis_index("c")
    base = wid * PER_W
    pltpu.sync_copy(idx_hbm.at[wid], idx_v)

    def step(j, _):
        pltpu.async_copy(tab_hbm.at[idx_v.at[j]], rows_v, gsem).wait()
        start = pl.multiple_of(base + j * CHUNK, CHUNK)
        pltpu.sync_copy(rows_v, out_hbm.at[pl.ds(start, CHUNK)])
        return 0

    lax.fori_loop(0, N_CHUNKS, step, 0)


def _sc_gather(idx, tab_flat, interpret=False):
    mesh = plsc.VectorSubcoreMesh(
        core_axis_name="c", subcore_axis_name="s",
        num_cores=NC, num_subcores=NS)
    return pl.kernel(
        _sc_gather_body,
        out_type=jax.ShapeDtypeStruct((N_ROWS, EMB), jnp.float32),
        mesh=mesh,
        scratch_types=[
            pltpu.VMEM((N_CHUNKS, CHUNK), jnp.int32),
            pltpu.VMEM((CHUNK, EMB), jnp.float32),
            pltpu.SemaphoreType.DMA,
        ],
        compiler_params=pltpu.CompilerParams(use_tc_tiling_on_sc=False),
        interpret=interpret,
    )(idx, tab_flat)


def _mlp_body(se_ref, xd_ref, w1s_ref, w1d_ref, b1_ref, w2_ref, b2_ref,
              w3_ref, b3_ref, ww_ref, bw_ref, mix_ref,
              logit_ref, prob_ref):
    se = se_ref[...]
    xd = xd_ref[...]
    h = jnp.dot(se, w1s_ref[...], preferred_element_type=jnp.float32)
    h += jnp.dot(xd, w1d_ref[...], preferred_element_type=jnp.float32)
    h = jnp.maximum(h + b1_ref[...], 0.0)
    h = jnp.maximum(
        jnp.dot(h, w2_ref[...], preferred_element_type=jnp.float32)
        + b2_ref[...], 0.0)
    deep = jnp.dot(h, w3_ref[...], preferred_element_type=jnp.float32) + b3_ref[...]
    wide = jnp.dot(xd, ww_ref[...], preferred_element_type=jnp.float32) + bw_ref[...]
    e = jnp.exp(mix_ref[...] - jnp.max(mix_ref[...]))
    w = e / jnp.sum(e)
    logit = wide * w[0:1, 0:1] + deep * w[0:1, 1:2]
    logit_ref[...] = logit
    prob_ref[...] = 1.0 / (1.0 + jnp.exp(-logit))


def _mlp(se, xd, w1s, w1d, b1, w2, b2, w3, b3, ww, bw, mix, interpret=False):
    BB = 2048
    grid = (B // BB,)
    const = lambda shape: pl.BlockSpec(shape, lambda i: (0, 0))
    return pl.pallas_call(
        _mlp_body,
        grid=grid,
        in_specs=[
            pl.BlockSpec((BB, SE_DIM), lambda i: (i, 0)),
            pl.BlockSpec((BB, DENSE), lambda i: (i, 0)),
            const((SE_DIM, 64)),
            const((DENSE, 64)),
            const((1, 64)),
            const((64, 32)),
            const((1, 32)),
            const((32, 1)),
            const((1, 1)),
            const((DENSE, 1)),
            const((1, 1)),
            const((1, 2)),
        ],
        out_specs=[
            pl.BlockSpec((BB, 1), lambda i: (i, 0)),
            pl.BlockSpec((BB, 1), lambda i: (i, 0)),
        ],
        out_shape=[
            jax.ShapeDtypeStruct((B, 1), jnp.float32),
            jax.ShapeDtypeStruct((B, 1), jnp.float32),
        ],
        interpret=interpret,
    )(se, xd, w1s, w1d, b1, w2, b2, w3, b3, ww, bw, mix)


@jax.jit
def kernel(x_sparse, x_dense, tables, W_wide, b_wide, W1, b1, W2, b2, W3, b3, mix):
    tab_flat = tables.reshape(F_SPARSE * VOCAB, EMB)
    offs = (jnp.arange(F_SPARSE, dtype=jnp.int32) * VOCAB)[None, :]
    idx = (x_sparse.astype(jnp.int32) + offs).reshape(NW, N_CHUNKS, CHUNK)
    se = _sc_gather(idx, tab_flat).reshape(B, SE_DIM)
    logit, prob = _mlp(
        se, x_dense,
        W1[:SE_DIM], W1[SE_DIM:], b1.reshape(1, 64),
        W2, b2.reshape(1, 32), W3, b3.reshape(1, 1),
        W_wide, b_wide.reshape(1, 1), mix.reshape(1, 2))
    return (logit, prob)
